# baseline (speedup 1.0000x reference)
import functools

import jax
import jax.numpy as jnp
from jax import lax
from jax.experimental import pallas as pl
from jax.experimental.pallas import tpu as pltpu
from jax.experimental.pallas import tpu_sc as plsc

NUM_CORES = 2
NUM_SUBCORES = 16
NUM_WORKERS = NUM_CORES * NUM_SUBCORES
VEC = 16


def kernel(x, pos_embedding, start_pos):
    batch, seq_len, d_model = x.shape
    sp = jnp.full((16,), start_pos, dtype=jnp.int32)

    rows_per_worker = seq_len // NUM_WORKERS
    chunk = min(8, rows_per_worker)
    n_chunks = rows_per_worker // chunk
    n_slots = 2 * batch
    n_iters = (n_chunks * batch) // n_slots
    vecs_per_row = d_model // VEC
    n_vecs = chunk * vecs_per_row
    chunk_mask = chunk - 1
    row_shift = vecs_per_row.bit_length() - 1
    batch_shift = batch.bit_length() - 1

    mesh = plsc.VectorSubcoreMesh(
        core_axis_name="c", subcore_axis_name="s",
        num_cores=NUM_CORES, num_subcores=NUM_SUBCORES)

    @functools.partial(
        pl.kernel,
        out_type=jax.ShapeDtypeStruct((batch, seq_len, d_model),
                                      jnp.float32),
        mesh=mesh,
        scratch_types=[
            pltpu.VMEM((16,), jnp.int32),
            [pltpu.VMEM((chunk, d_model), jnp.float32)] * 2,
            [pltpu.VMEM((chunk, d_model), jnp.float32)] * n_slots,
            [pltpu.SemaphoreType.DMA] * 2,
            [pltpu.SemaphoreType.DMA] * n_slots,
            [pltpu.SemaphoreType.DMA] * n_slots,
        ],
        compiler_params=pltpu.CompilerParams(use_tc_tiling_on_sc=True),
    )
    def run(x_hbm, pos_hbm, sp_hbm, out_hbm, sp_vmem, posbufs, xbufs,
            pos_sems, ld_sems, st_sems):
        cid = lax.axis_index("c")
        sid = lax.axis_index("s")
        wid = sid * NUM_CORES + cid
        pltpu.sync_copy(sp_hbm, sp_vmem)
        s0 = sp_vmem[...][0]
        base = wid * rows_per_worker

        def rows_of(c):
            return pl.multiple_of(base + c * chunk, chunk)

        def issue_pos(c, slot):
            prow = pl.multiple_of(s0 + rows_of(c), 8)
            pltpu.async_copy(pos_hbm.at[pl.ds(prow, chunk)],
                             posbufs[slot], pos_sems[slot])

        def issue_ld(t, b, slot):
            c = lax.shift_right_logical(t, batch_shift)
            pltpu.async_copy(
                x_hbm.at[b, pl.ds(rows_of(c), chunk)],
                xbufs[slot], ld_sems[slot])

        def drain_ld(slot):
            pltpu.make_async_copy(
                x_hbm.at[0, pl.ds(0, chunk)],
                xbufs[slot], ld_sems[slot]).wait()

        def drain_st(slot):
            pltpu.make_async_copy(
                xbufs[slot], out_hbm.at[0, pl.ds(0, chunk)],
                st_sems[slot]).wait()

        def drain_pos(slot):
            pltpu.make_async_copy(
                pos_hbm.at[pl.ds(0, chunk)], posbufs[slot],
                pos_sems[slot]).wait()

        issue_pos(0, 0)
        issue_pos(1, 1)
        for b in range(batch):
            issue_ld(b, b, b)

        half = n_slots // 2

        def iteration(k, _):
            for j in range(n_slots):
                t = k * n_slots + j
                b = j % batch
                pslot = (j >= half) * 1
                if j % half == 0:
                    drain_pos(pslot)
                refill = (j + half) % n_slots
                if j < half:

                    @pl.when(k >= 1)
                    def _():
                        drain_st(refill)

                    issue_ld(t + half, b, refill)
                else:
                    drain_st(refill)

                    @pl.when(k < n_iters - 1)
                    def _():
                        issue_ld(t + half, b, refill)

                drain_ld(j)
                xbuf, posbuf = xbufs[j], posbufs[pslot]

                @plsc.parallel_loop(0, n_vecs, 1, unroll=8)
                def body(i):
                    r = lax.bitwise_and(
                        lax.shift_right_logical(i, row_shift), chunk_mask)
                    col = lax.mul(lax.bitwise_and(i, vecs_per_row - 1),
                                  VEC)
                    xbuf[r, pl.ds(col, VEC)] = (
                        xbuf[r, pl.ds(col, VEC)]
                        + posbuf[r, pl.ds(col, VEC)])

                c = lax.shift_right_logical(t, batch_shift)
                pltpu.async_copy(
                    xbufs[j], out_hbm.at[b, pl.ds(rows_of(c), chunk)],
                    st_sems[j])
                if j % half == half - 1:

                    @pl.when(c + 2 < n_chunks)
                    def _():
                        issue_pos(c + 2, pslot)

            return 0

        lax.fori_loop(0, n_iters, iteration, 0)
        for s in range(half, n_slots):
            drain_st(s)

    return run(x, pos_embedding, sp)
